# Initial kernel scaffold; baseline (speedup 1.0000x reference)
import jax
import jax.numpy as jnp
from jax.experimental import pallas as pl


def kernel(x, adj, e, weights):
    raise NotImplementedError("write your pallas kernel here")


# speedup vs baseline: 14.8091x; 14.8091x over previous
import functools

import jax
import jax.numpy as jnp
from jax import lax
from jax.experimental import pallas as pl
from jax.experimental.pallas import tpu as pltpu
from jax.experimental.pallas import tpu_sc as plsc

N = 10000
E = 320000
F = 128
K = 4

NC = 2
NS = 16
NW = NC * NS

CHUNK = 128
NCH = E // CHUNK
FULL_ITERS = NCH // NW
REM = NCH - FULL_ITERS * NW

NPAD = 10240
HWORDS = NPAD // NS
ROWS_PER_TILE = NPAD // NS
ZR = 80
CR = 128

_mesh = plsc.VectorSubcoreMesh(core_axis_name="c", subcore_axis_name="s",
                               num_cores=NC, num_subcores=NS)


@functools.partial(
    pl.kernel,
    out_type=jax.ShapeDtypeStruct((NC, NPAD), jnp.float32),
    mesh=_mesh,
    scratch_types=[
        pltpu.VMEM((2, CHUNK), jnp.int32),
        pltpu.VMEM((CHUNK,), jnp.float32),
        pltpu.VMEM((HWORDS,), jnp.float32),
        pltpu.VMEM_SHARED((NPAD,), jnp.float32),
        pltpu.SemaphoreType.DMA,
    ],
)
def _hist_kernel(col2_hbm, ones_hbm, zeros_hbm, out_hbm,
                 cidx, ones_v, zb, hist_sh, sem):
    cid = lax.axis_index("c")
    sid = lax.axis_index("s")
    w = sid * NC + cid

    pltpu.sync_copy(ones_hbm, ones_v)
    pltpu.sync_copy(zeros_hbm, zb)
    pltpu.sync_copy(zb, hist_sh.at[pl.ds(sid * HWORDS, HWORDS)])
    plsc.subcore_barrier()

    def step(j):
        b = j & 1
        c = w + NW * j
        pltpu.sync_copy(col2_hbm.at[c], cidx.at[b])
        pltpu.sync_copy(ones_v, hist_sh.at[cidx.at[b]], add=True)

    for j in range(FULL_ITERS):
        step(j)

    @pl.when(w < REM)
    def _():
        step(FULL_ITERS)

    plsc.subcore_barrier()
    pltpu.sync_copy(hist_sh.at[pl.ds(sid * HWORDS, HWORDS)], zb)
    pltpu.sync_copy(zb, out_hbm.at[cid, pl.ds(sid * HWORDS, HWORDS)])


@functools.partial(
    pl.kernel,
    out_type=jax.ShapeDtypeStruct((NC, NPAD, F), jnp.float32),
    mesh=_mesh,
    scratch_types=[
        pltpu.VMEM((2, CHUNK), jnp.int32),
        pltpu.VMEM((2, CHUNK), jnp.int32),
        pltpu.VMEM((2, CHUNK, F), jnp.float32),
        pltpu.VMEM_SHARED((NPAD, F), jnp.float32),
        pltpu.SemaphoreType.DMA,
        pltpu.SemaphoreType.DMA,
    ],
)
def _spmm_kernel(xs_hbm, row2_hbm, col2_hbm, zeros_hbm, out_hbm,
                 ridx, cidx, gbuf, acc_sh, sem0, sem1):
    cid = lax.axis_index("c")
    sid = lax.axis_index("s")
    w = sid * NC + cid
    sems = (sem0, sem1)

    pltpu.sync_copy(zeros_hbm, gbuf.at[0])
    base = sid * ROWS_PER_TILE
    for t in range(ROWS_PER_TILE // CR):
        pltpu.sync_copy(gbuf.at[0], acc_sh.at[pl.ds(base + t * CR, CR)])
    plsc.subcore_barrier()

    def step(j):
        b = j & 1
        c = w + NW * j
        pltpu.sync_copy(row2_hbm.at[c], ridx.at[b])
        pltpu.sync_copy(col2_hbm.at[c], cidx.at[b])
        pltpu.async_copy(xs_hbm.at[ridx.at[b]], gbuf.at[b], sems[b]).wait()
        pltpu.sync_copy(gbuf.at[b], acc_sh.at[cidx.at[b]], add=True)

    for j in range(FULL_ITERS):
        step(j)

    @pl.when(w < REM)
    def _():
        step(FULL_ITERS)

    plsc.subcore_barrier()
    for t in range(ROWS_PER_TILE // CR):
        r = base + t * CR
        pltpu.sync_copy(acc_sh.at[pl.ds(r, CR)], gbuf.at[0])
        pltpu.sync_copy(gbuf.at[0], out_hbm.at[cid, pl.ds(r, CR)])


def _scale_body(dp_ref, x_ref, s_ref, xs_ref):
    d = dp_ref[0, :, :] + dp_ref[1, :, :]
    s = jnp.where(d > 0.0, lax.rsqrt(jnp.maximum(d, 1.0)), 0.0)
    s_ref[...] = s
    xs_ref[...] = s * x_ref[...]


def _dense_body(acc_ref, s_ref, x_ref, e_ref, w_ref, o_ref):
    g = s_ref[...] * (acc_ref[0, :, :] + acc_ref[1, :, :])
    xb = x_ref[...]
    out = xb
    for k in range(K):
        y = lax.dot_general(g, w_ref[k, 0:F, :], (((1,), (0,)), ((), ())),
                            preferred_element_type=jnp.float32)
        y = y + lax.dot_general(xb, w_ref[k, F:2 * F, :],
                                (((1,), (0,)), ((), ())),
                                preferred_element_type=jnp.float32)
        out = out + e_ref[:, k:k + 1] * y
    o_ref[...] = out


_BR = 1000


def kernel(x, adj, e, weights):
    row2 = adj[0].reshape(NCH, CHUNK)
    col2 = adj[1].reshape(NCH, CHUNK)

    ones1 = jnp.ones((CHUNK,), jnp.float32)
    zeros1 = jnp.zeros((HWORDS,), jnp.float32)
    zeros2 = jnp.zeros((CR, F), jnp.float32)

    dp = _hist_kernel(col2, ones1, zeros1)
    dp3 = dp[:, :N].reshape(NC, N, 1)

    s, xs = pl.pallas_call(
        _scale_body,
        out_shape=(
            jax.ShapeDtypeStruct((N, 1), jnp.float32),
            jax.ShapeDtypeStruct((N, F), jnp.float32),
        ),
    )(dp3, x)

    acc = _spmm_kernel(xs, row2, col2, zeros2)

    out = pl.pallas_call(
        _dense_body,
        grid=(N // _BR,),
        in_specs=[
            pl.BlockSpec((NC, _BR, F), lambda i: (0, i, 0)),
            pl.BlockSpec((_BR, 1), lambda i: (i, 0)),
            pl.BlockSpec((_BR, F), lambda i: (i, 0)),
            pl.BlockSpec((_BR, K), lambda i: (i, 0)),
            pl.BlockSpec((K, 2 * F, F), lambda i: (0, 0, 0)),
        ],
        out_specs=pl.BlockSpec((_BR, F), lambda i: (i, 0)),
        out_shape=jax.ShapeDtypeStruct((N, F), jnp.float32),
    )(acc, s, x, e, weights)
    return out


# speedup vs baseline: 24.6660x; 1.6656x over previous
import functools

import jax
import jax.numpy as jnp
from jax import lax
from jax.experimental import pallas as pl
from jax.experimental.pallas import tpu as pltpu
from jax.experimental.pallas import tpu_sc as plsc

N = 10000
E = 320000
F = 128
K = 4

NC = 2
NS = 16
NW = NC * NS

CHUNK = 128
NCH = E // CHUNK
FULL_ITERS = NCH // NW
REM = NCH - FULL_ITERS * NW

NPAD = 10240
HWORDS = NPAD // NS
ROWS_PER_TILE = NPAD // NS
ZR = 80
CR = 128

_mesh = plsc.VectorSubcoreMesh(core_axis_name="c", subcore_axis_name="s",
                               num_cores=NC, num_subcores=NS)


@functools.partial(
    pl.kernel,
    out_type=jax.ShapeDtypeStruct((NC, NPAD), jnp.float32),
    mesh=_mesh,
    scratch_types=[
        pltpu.VMEM((2, CHUNK), jnp.int32),
        pltpu.VMEM((CHUNK,), jnp.float32),
        pltpu.VMEM((HWORDS,), jnp.float32),
        pltpu.VMEM_SHARED((NPAD,), jnp.float32),
        pltpu.SemaphoreType.DMA,
        pltpu.SemaphoreType.DMA,
    ],
)
def _hist_kernel(col2_hbm, ones_hbm, zeros_hbm, out_hbm,
                 cidx, ones_v, zb, hist_sh, semi0, semi1):
    cid = lax.axis_index("c")
    sid = lax.axis_index("s")
    w = sid * NC + cid
    semi = (semi0, semi1)
    niter = FULL_ITERS + 1

    pltpu.sync_copy(ones_hbm, ones_v)
    pltpu.sync_copy(zeros_hbm, zb)
    pltpu.sync_copy(zb, hist_sh.at[pl.ds(sid * HWORDS, HWORDS)])
    plsc.subcore_barrier()

    pltpu.sync_copy(col2_hbm.at[w], cidx.at[0])
    for j in range(niter):
        b = j & 1
        nb = 1 - b
        if j + 1 < niter:
            start = lambda jn=j + 1, nb=nb: pltpu.async_copy(
                col2_hbm.at[w + NW * jn], cidx.at[nb], semi[nb])
            if j + 1 == niter - 1:
                pl.when(w < REM)(lambda: start() and None)
            else:
                start()
        scat = lambda b=b: pltpu.sync_copy(ones_v, hist_sh.at[cidx.at[b]],
                                           add=True)
        wait = lambda jn=j, b=b, nb=nb: pltpu.make_async_copy(
            col2_hbm.at[w + NW * jn], cidx.at[b], semi[b]).wait()
        if j == niter - 1:
            @pl.when(w < REM)
            def _(wait=wait, scat=scat):
                wait()
                scat()
        else:
            if j > 0:
                wait()
            scat()

    plsc.subcore_barrier()
    pltpu.sync_copy(hist_sh.at[pl.ds(sid * HWORDS, HWORDS)], zb)
    pltpu.sync_copy(zb, out_hbm.at[cid, pl.ds(sid * HWORDS, HWORDS)])


@functools.partial(
    pl.kernel,
    out_type=jax.ShapeDtypeStruct((NC, NPAD, F), jnp.float32),
    mesh=_mesh,
    scratch_types=[
        pltpu.VMEM((2, CHUNK), jnp.int32),
        pltpu.VMEM((2, CHUNK), jnp.int32),
        pltpu.VMEM((2, CHUNK, F), jnp.float32),
        pltpu.VMEM_SHARED((NPAD, F), jnp.float32),
        pltpu.SemaphoreType.DMA,
        pltpu.SemaphoreType.DMA,
        pltpu.SemaphoreType.DMA,
        pltpu.SemaphoreType.DMA,
        pltpu.SemaphoreType.DMA,
        pltpu.SemaphoreType.DMA,
    ],
)
def _spmm_kernel(xs_hbm, row2_hbm, col2_hbm, zeros_hbm, out_hbm,
                 ridx, cidx, gbuf, acc_sh,
                 semg0, semg1, semr0, semr1, semc0, semc1):
    cid = lax.axis_index("c")
    sid = lax.axis_index("s")
    w = sid * NC + cid
    semg = (semg0, semg1)
    semr = (semr0, semr1)
    semc = (semc0, semc1)
    niter = FULL_ITERS + 1

    pltpu.sync_copy(zeros_hbm, gbuf.at[0])
    base = sid * ROWS_PER_TILE
    for t in range(ROWS_PER_TILE // CR):
        pltpu.sync_copy(gbuf.at[0], acc_sh.at[pl.ds(base + t * CR, CR)])
    plsc.subcore_barrier()

    def stage(j, b):
        pltpu.async_copy(row2_hbm.at[w + NW * j], ridx.at[b], semr[b])
        pltpu.async_copy(col2_hbm.at[w + NW * j], cidx.at[b], semc[b])

    def wait_stage(j, b):
        pltpu.make_async_copy(row2_hbm.at[w + NW * j], ridx.at[b],
                              semr[b]).wait()
        pltpu.make_async_copy(col2_hbm.at[w + NW * j], cidx.at[b],
                              semc[b]).wait()

    def gather(b):
        pltpu.async_copy(xs_hbm.at[ridx.at[b]], gbuf.at[b], semg[b])

    def wait_gather_scatter(b):
        pltpu.make_async_copy(xs_hbm.at[ridx.at[b]], gbuf.at[b],
                              semg[b]).wait()
        pltpu.sync_copy(gbuf.at[b], acc_sh.at[cidx.at[b]], add=True)

    pltpu.sync_copy(row2_hbm.at[w], ridx.at[0])
    pltpu.sync_copy(col2_hbm.at[w], cidx.at[0])
    for j in range(niter):
        b = j & 1
        nb = 1 - b
        if j == niter - 1:
            pl.when(w < REM)(lambda b=b: gather(b))
        else:
            gather(b)
        if j > 0:
            wait_gather_scatter(nb)
        if j + 1 < niter:
            if j + 1 == niter - 1:
                pl.when(w < REM)(lambda jn=j + 1, nb=nb: stage(jn, nb))
                pl.when(w < REM)(lambda jn=j + 1, nb=nb: wait_stage(jn, nb))
            else:
                stage(j + 1, nb)
                wait_stage(j + 1, nb)
    last = niter - 1
    pl.when(w < REM)(lambda b=last & 1: wait_gather_scatter(b))

    plsc.subcore_barrier()
    nco = ROWS_PER_TILE // CR
    for t in range(nco):
        b = t & 1
        r = base + t * CR
        if t >= 2:
            pr = base + (t - 2) * CR
            pltpu.make_async_copy(gbuf.at[b],
                                  out_hbm.at[cid, pl.ds(pr, CR)],
                                  semg[b]).wait()
        pltpu.sync_copy(acc_sh.at[pl.ds(r, CR)], gbuf.at[b])
        pltpu.async_copy(gbuf.at[b], out_hbm.at[cid, pl.ds(r, CR)], semg[b])
    for t in range(nco - 2, nco):
        b = t & 1
        r = base + t * CR
        pltpu.make_async_copy(gbuf.at[b], out_hbm.at[cid, pl.ds(r, CR)],
                              semg[b]).wait()


def _scale_body(dp_ref, x_ref, s_ref, xs_ref):
    d = dp_ref[0, :, :] + dp_ref[1, :, :]
    s = jnp.where(d > 0.0, lax.rsqrt(jnp.maximum(d, 1.0)), 0.0)
    s_ref[...] = s
    xs_ref[...] = s * x_ref[...]


def _dense_body(acc_ref, s_ref, x_ref, e_ref, w_ref, o_ref):
    g = s_ref[...] * (acc_ref[0, :, :] + acc_ref[1, :, :])
    xb = x_ref[...]
    out = xb
    for k in range(K):
        y = lax.dot_general(g, w_ref[k, 0:F, :], (((1,), (0,)), ((), ())),
                            preferred_element_type=jnp.float32)
        y = y + lax.dot_general(xb, w_ref[k, F:2 * F, :],
                                (((1,), (0,)), ((), ())),
                                preferred_element_type=jnp.float32)
        out = out + e_ref[:, k:k + 1] * y
    o_ref[...] = out


_BR = 1000


def kernel(x, adj, e, weights):
    row2 = adj[0].reshape(NCH, CHUNK)
    col2 = adj[1].reshape(NCH, CHUNK)

    ones1 = jnp.ones((CHUNK,), jnp.float32)
    zeros1 = jnp.zeros((HWORDS,), jnp.float32)
    zeros2 = jnp.zeros((CR, F), jnp.float32)

    dp = _hist_kernel(col2, ones1, zeros1)
    dp3 = dp[:, :N].reshape(NC, N, 1)

    s, xs = pl.pallas_call(
        _scale_body,
        out_shape=(
            jax.ShapeDtypeStruct((N, 1), jnp.float32),
            jax.ShapeDtypeStruct((N, F), jnp.float32),
        ),
    )(dp3, x)

    acc = _spmm_kernel(xs, row2, col2, zeros2)

    out = pl.pallas_call(
        _dense_body,
        grid=(N // _BR,),
        in_specs=[
            pl.BlockSpec((NC, _BR, F), lambda i: (0, i, 0)),
            pl.BlockSpec((_BR, 1), lambda i: (i, 0)),
            pl.BlockSpec((_BR, F), lambda i: (i, 0)),
            pl.BlockSpec((_BR, K), lambda i: (i, 0)),
            pl.BlockSpec((K, 2 * F, F), lambda i: (0, 0, 0)),
        ],
        out_specs=pl.BlockSpec((_BR, F), lambda i: (i, 0)),
        out_shape=jax.ShapeDtypeStruct((N, F), jnp.float32),
    )(acc, s, x, e, weights)
    return out


# speedup vs baseline: 25.4545x; 1.0320x over previous
import functools

import jax
import jax.numpy as jnp
from jax import lax
from jax.experimental import pallas as pl
from jax.experimental.pallas import tpu as pltpu
from jax.experimental.pallas import tpu_sc as plsc

N = 10000
E = 320000
F = 128
K = 4

NC = 2
NS = 16
NW = NC * NS

CHUNK = 128
NCH = E // CHUNK
FULL_ITERS = NCH // NW
REM = NCH - FULL_ITERS * NW

NPAD = 10112
HWORDS = NPAD // NS
ROWS_PER_TILE = NPAD // NS
CR = 128
CO_CHUNKS = (CR, CR, CR, CR, ROWS_PER_TILE - 4 * CR)
NBUF = 3

_mesh = plsc.VectorSubcoreMesh(core_axis_name="c", subcore_axis_name="s",
                               num_cores=NC, num_subcores=NS)


@functools.partial(
    pl.kernel,
    out_type=jax.ShapeDtypeStruct((NC * NPAD,), jnp.float32),
    mesh=_mesh,
    scratch_types=[
        pltpu.VMEM((2, CHUNK), jnp.int32),
        pltpu.VMEM((CHUNK,), jnp.float32),
        pltpu.VMEM((HWORDS,), jnp.float32),
        pltpu.VMEM_SHARED((NPAD,), jnp.float32),
        pltpu.SemaphoreType.DMA,
        pltpu.SemaphoreType.DMA,
    ],
)
def _hist_kernel(col2_hbm, ones_hbm, zeros_hbm, out_hbm,
                 cidx, ones_v, zb, hist_sh, semi0, semi1):
    cid = lax.axis_index("c")
    sid = lax.axis_index("s")
    w = sid * NC + cid
    semi = (semi0, semi1)
    niter = FULL_ITERS + 1

    pltpu.sync_copy(ones_hbm, ones_v)
    pltpu.sync_copy(zeros_hbm, zb)
    pltpu.sync_copy(zb, hist_sh.at[pl.ds(sid * HWORDS, HWORDS)])
    plsc.subcore_barrier()

    pltpu.sync_copy(col2_hbm.at[w], cidx.at[0])
    for j in range(niter):
        b = j & 1
        nb = 1 - b
        if j + 1 < niter:
            start = lambda jn=j + 1, nb=nb: pltpu.async_copy(
                col2_hbm.at[w + NW * jn], cidx.at[nb], semi[nb])
            if j + 1 == niter - 1:
                pl.when(w < REM)(lambda: start() and None)
            else:
                start()
        scat = lambda b=b: pltpu.sync_copy(ones_v, hist_sh.at[cidx.at[b]],
                                           add=True)
        wait = lambda jn=j, b=b, nb=nb: pltpu.make_async_copy(
            col2_hbm.at[w + NW * jn], cidx.at[b], semi[b]).wait()
        if j == niter - 1:
            @pl.when(w < REM)
            def _(wait=wait, scat=scat):
                wait()
                scat()
        else:
            if j > 0:
                wait()
            scat()

    plsc.subcore_barrier()
    pltpu.sync_copy(hist_sh.at[pl.ds(sid * HWORDS, HWORDS)], zb)
    pltpu.sync_copy(zb, out_hbm.at[pl.ds(cid * NPAD + sid * HWORDS, HWORDS)])


@functools.partial(
    pl.kernel,
    out_type=jax.ShapeDtypeStruct((NC, NPAD, F), jnp.float32),
    mesh=_mesh,
    scratch_types=[
        pltpu.VMEM((NBUF, CHUNK), jnp.int32),
        pltpu.VMEM((NBUF, CHUNK), jnp.int32),
        pltpu.VMEM((NBUF, CHUNK, F), jnp.float32),
        pltpu.VMEM_SHARED((NPAD, F), jnp.float32),
        [pltpu.SemaphoreType.DMA] * NBUF,
        [pltpu.SemaphoreType.DMA] * NBUF,
        [pltpu.SemaphoreType.DMA] * NBUF,
    ],
)
def _spmm_kernel(xs_hbm, row2_hbm, col2_hbm, zeros_hbm, out_hbm,
                 ridx, cidx, gbuf, acc_sh, semg, semr, semc):
    cid = lax.axis_index("c")
    sid = lax.axis_index("s")
    w = sid * NC + cid
    niter = FULL_ITERS + 1

    pltpu.sync_copy(zeros_hbm, gbuf.at[0])
    base = sid * ROWS_PER_TILE
    r0 = 0
    for sz in CO_CHUNKS:
        pltpu.sync_copy(gbuf.at[0, pl.ds(0, sz)],
                        acc_sh.at[pl.ds(base + r0, sz)])
        r0 += sz
    plsc.subcore_barrier()

    def stage(j, b):
        pltpu.async_copy(row2_hbm.at[w + NW * j], ridx.at[b], semr[b])
        pltpu.async_copy(col2_hbm.at[w + NW * j], cidx.at[b], semc[b])

    def wait_stage(j, b):
        pltpu.make_async_copy(row2_hbm.at[w + NW * j], ridx.at[b],
                              semr[b]).wait()
        pltpu.make_async_copy(col2_hbm.at[w + NW * j], cidx.at[b],
                              semc[b]).wait()

    def gather(b):
        pltpu.async_copy(xs_hbm.at[ridx.at[b]], gbuf.at[b], semg[b])

    def wait_gather_scatter(b):
        pltpu.make_async_copy(xs_hbm.at[ridx.at[b]], gbuf.at[b],
                              semg[b]).wait()
        pltpu.sync_copy(gbuf.at[b], acc_sh.at[cidx.at[b]], add=True)

    pltpu.sync_copy(row2_hbm.at[w], ridx.at[0])
    pltpu.sync_copy(col2_hbm.at[w], cidx.at[0])
    for j in range(niter):
        b = j % NBUF
        if j == niter - 1:
            pl.when(w < REM)(lambda b=b: gather(b))
        else:
            gather(b)
        if j >= 2:
            wait_gather_scatter((j - 2) % NBUF)
        if j + 1 < niter:
            nb = (j + 1) % NBUF
            if j + 1 == niter - 1:
                pl.when(w < REM)(lambda jn=j + 1, nb=nb: stage(jn, nb))
                pl.when(w < REM)(lambda jn=j + 1, nb=nb: wait_stage(jn, nb))
            else:
                stage(j + 1, nb)
                wait_stage(j + 1, nb)
    wait_gather_scatter((niter - 2) % NBUF)
    pl.when(w < REM)(lambda b=(niter - 1) % NBUF: wait_gather_scatter(b))

    plsc.subcore_barrier()
    nco = len(CO_CHUNKS)
    offs = [base + sum(CO_CHUNKS[:t]) for t in range(nco)]
    for t in range(nco):
        b = t % NBUF
        r, sz = offs[t], CO_CHUNKS[t]
        if t >= NBUF:
            pr, psz = offs[t - NBUF], CO_CHUNKS[t - NBUF]
            pltpu.make_async_copy(gbuf.at[b, pl.ds(0, psz)],
                                  out_hbm.at[cid, pl.ds(pr, psz)],
                                  semg[b]).wait()
        pltpu.sync_copy(acc_sh.at[pl.ds(r, sz)], gbuf.at[b, pl.ds(0, sz)])
        pltpu.async_copy(gbuf.at[b, pl.ds(0, sz)],
                         out_hbm.at[cid, pl.ds(r, sz)], semg[b])
    for t in range(max(0, nco - NBUF), nco):
        b = t % NBUF
        r, sz = offs[t], CO_CHUNKS[t]
        pltpu.make_async_copy(gbuf.at[b, pl.ds(0, sz)],
                              out_hbm.at[cid, pl.ds(r, sz)], semg[b]).wait()


def _scale_body(dp_ref, x_ref, s_ref, xs_ref):
    d = dp_ref[0, :, :] + dp_ref[1, :, :]
    s = jnp.where(d > 0.0, lax.rsqrt(jnp.maximum(d, 1.0)), 0.0)
    s_ref[...] = s
    xs_ref[...] = s * x_ref[...]


def _dense_body(acc_ref, s_ref, x_ref, e_ref, w_ref, o_ref):
    g = s_ref[...] * (acc_ref[0, :, :] + acc_ref[1, :, :])
    xb = x_ref[...]
    y = lax.dot_general(g, w_ref[0:F, :], (((1,), (0,)), ((), ())),
                        preferred_element_type=jnp.float32)
    y = y + lax.dot_general(xb, w_ref[F:2 * F, :], (((1,), (0,)), ((), ())),
                            preferred_element_type=jnp.float32)
    out = xb
    for k in range(K):
        out = out + e_ref[:, k:k + 1] * y[:, k * F:(k + 1) * F]
    o_ref[...] = out


_BR = 1000


def kernel(x, adj, e, weights):
    row2 = adj[0].reshape(NCH, CHUNK)
    col2 = adj[1].reshape(NCH, CHUNK)

    ones1 = jnp.ones((CHUNK,), jnp.float32)
    zeros1 = jnp.zeros((HWORDS,), jnp.float32)
    zeros2 = jnp.zeros((CR, F), jnp.float32)
    wall = jnp.transpose(weights, (1, 0, 2)).reshape(2 * F, K * F)

    dp = _hist_kernel(col2, ones1, zeros1).reshape(NC, NPAD)
    dp3 = dp[:, :N].reshape(NC, N, 1)

    s, xs = pl.pallas_call(
        _scale_body,
        out_shape=(
            jax.ShapeDtypeStruct((N, 1), jnp.float32),
            jax.ShapeDtypeStruct((N, F), jnp.float32),
        ),
    )(dp3, x)

    acc = _spmm_kernel(xs, row2, col2, zeros2)

    out = pl.pallas_call(
        _dense_body,
        grid=(N // _BR,),
        in_specs=[
            pl.BlockSpec((NC, _BR, F), lambda i: (0, i, 0)),
            pl.BlockSpec((_BR, 1), lambda i: (i, 0)),
            pl.BlockSpec((_BR, F), lambda i: (i, 0)),
            pl.BlockSpec((_BR, K), lambda i: (i, 0)),
            pl.BlockSpec((2 * F, K * F), lambda i: (0, 0)),
        ],
        out_specs=pl.BlockSpec((_BR, F), lambda i: (i, 0)),
        out_shape=jax.ShapeDtypeStruct((N, F), jnp.float32),
    )(acc, s, x, e, wall)
    return out


# speedup vs baseline: 27.6595x; 1.0866x over previous
import functools

import jax
import jax.numpy as jnp
from jax import lax
from jax.experimental import pallas as pl
from jax.experimental.pallas import tpu as pltpu
from jax.experimental.pallas import tpu_sc as plsc

N = 10000
E = 320000
F = 128
K = 4

NC = 2
NS = 16
NW = NC * NS

CHUNK = 128
NCH = E // CHUNK
NCHP = 2560
FULL_ITERS = NCH // NW
REM = NCH - FULL_ITERS * NW

NPAD = 10112
HWORDS = NPAD // NS
ROWS_PER_TILE = NPAD // NS
CR = 128
CO_CHUNKS = (CR, CR, CR, CR, ROWS_PER_TILE - 4 * CR)
NBUF = 3

_mesh = plsc.VectorSubcoreMesh(core_axis_name="c", subcore_axis_name="s",
                               num_cores=NC, num_subcores=NS)


HCHW = NCHP // NW


@functools.partial(
    pl.kernel,
    out_type=jax.ShapeDtypeStruct((NC * NPAD,), jnp.float32),
    mesh=_mesh,
    scratch_types=[
        pltpu.VMEM((HCHW, CHUNK), jnp.int32),
        pltpu.VMEM((CHUNK,), jnp.float32),
        pltpu.VMEM((HWORDS,), jnp.float32),
        pltpu.VMEM_SHARED((NPAD,), jnp.float32),
        pltpu.SemaphoreType.DMA,
    ],
)
def _hist_kernel(colp_hbm, ones_hbm, zeros_hbm, out_hbm,
                 cidx, ones_v, zb, hist_sh, semi):
    cid = lax.axis_index("c")
    sid = lax.axis_index("s")
    w = sid * NC + cid

    pltpu.async_copy(colp_hbm.at[pl.ds(w * HCHW, HCHW)], cidx, semi)
    pltpu.sync_copy(ones_hbm, ones_v)
    pltpu.sync_copy(zeros_hbm, zb)
    pltpu.sync_copy(zb, hist_sh.at[pl.ds(sid * HWORDS, HWORDS)])
    pltpu.make_async_copy(colp_hbm.at[pl.ds(w * HCHW, HCHW)], cidx,
                          semi).wait()
    plsc.subcore_barrier()

    for j in range(HCHW):
        @pl.when(w * HCHW + j < NCH)
        def _(j=j):
            pltpu.sync_copy(ones_v, hist_sh.at[cidx.at[j]], add=True)

    plsc.subcore_barrier()
    pltpu.sync_copy(hist_sh.at[pl.ds(sid * HWORDS, HWORDS)], zb)
    pltpu.sync_copy(zb, out_hbm.at[pl.ds(cid * NPAD + sid * HWORDS, HWORDS)])


@functools.partial(
    pl.kernel,
    out_type=jax.ShapeDtypeStruct((NC, NPAD, F), jnp.float32),
    mesh=_mesh,
    scratch_types=[
        pltpu.VMEM((NBUF, CHUNK), jnp.int32),
        pltpu.VMEM((NBUF, CHUNK), jnp.int32),
        pltpu.VMEM((NBUF, CHUNK, F), jnp.float32),
        pltpu.VMEM_SHARED((NPAD, F), jnp.float32),
        [pltpu.SemaphoreType.DMA] * NBUF,
        [pltpu.SemaphoreType.DMA] * NBUF,
        [pltpu.SemaphoreType.DMA] * NBUF,
    ],
)
def _spmm_kernel(xs_hbm, row2_hbm, col2_hbm, zeros_hbm, out_hbm,
                 ridx, cidx, gbuf, acc_sh, semg, semr, semc):
    cid = lax.axis_index("c")
    sid = lax.axis_index("s")
    w = sid * NC + cid
    niter = FULL_ITERS + 1

    pltpu.sync_copy(zeros_hbm, gbuf.at[0])
    base = sid * ROWS_PER_TILE
    r0 = 0
    for sz in CO_CHUNKS:
        pltpu.sync_copy(gbuf.at[0, pl.ds(0, sz)],
                        acc_sh.at[pl.ds(base + r0, sz)])
        r0 += sz
    plsc.subcore_barrier()

    def stage(j, b):
        pltpu.async_copy(row2_hbm.at[w + NW * j], ridx.at[b], semr[b])
        pltpu.async_copy(col2_hbm.at[w + NW * j], cidx.at[b], semc[b])

    def wait_stage(j, b):
        pltpu.make_async_copy(row2_hbm.at[w + NW * j], ridx.at[b],
                              semr[b]).wait()
        pltpu.make_async_copy(col2_hbm.at[w + NW * j], cidx.at[b],
                              semc[b]).wait()

    def gather(b):
        pltpu.async_copy(xs_hbm.at[ridx.at[b]], gbuf.at[b], semg[b])

    def wait_gather_scatter(b):
        pltpu.make_async_copy(xs_hbm.at[ridx.at[b]], gbuf.at[b],
                              semg[b]).wait()
        pltpu.sync_copy(gbuf.at[b], acc_sh.at[cidx.at[b]], add=True)

    pltpu.sync_copy(row2_hbm.at[w], ridx.at[0])
    pltpu.sync_copy(col2_hbm.at[w], cidx.at[0])
    for j in range(niter):
        b = j % NBUF
        if j == niter - 1:
            pl.when(w < REM)(lambda b=b: gather(b))
        else:
            gather(b)
        if j >= 2:
            wait_gather_scatter((j - 2) % NBUF)
        if j + 1 < niter:
            nb = (j + 1) % NBUF
            if j + 1 == niter - 1:
                pl.when(w < REM)(lambda jn=j + 1, nb=nb: stage(jn, nb))
                pl.when(w < REM)(lambda jn=j + 1, nb=nb: wait_stage(jn, nb))
            else:
                stage(j + 1, nb)
                wait_stage(j + 1, nb)
    wait_gather_scatter((niter - 2) % NBUF)
    pl.when(w < REM)(lambda b=(niter - 1) % NBUF: wait_gather_scatter(b))

    plsc.subcore_barrier()
    nco = len(CO_CHUNKS)
    offs = [base + sum(CO_CHUNKS[:t]) for t in range(nco)]
    for t in range(nco):
        b = t % NBUF
        r, sz = offs[t], CO_CHUNKS[t]
        if t >= NBUF:
            pr, psz = offs[t - NBUF], CO_CHUNKS[t - NBUF]
            pltpu.make_async_copy(gbuf.at[b, pl.ds(0, psz)],
                                  out_hbm.at[cid, pl.ds(pr, psz)],
                                  semg[b]).wait()
        pltpu.sync_copy(acc_sh.at[pl.ds(r, sz)], gbuf.at[b, pl.ds(0, sz)])
        pltpu.async_copy(gbuf.at[b, pl.ds(0, sz)],
                         out_hbm.at[cid, pl.ds(r, sz)], semg[b])
    for t in range(max(0, nco - NBUF), nco):
        b = t % NBUF
        r, sz = offs[t], CO_CHUNKS[t]
        pltpu.make_async_copy(gbuf.at[b, pl.ds(0, sz)],
                              out_hbm.at[cid, pl.ds(r, sz)], semg[b]).wait()


def _scale_body(dp_ref, x_ref, s_ref, xs_ref):
    d = dp_ref[0, :, :] + dp_ref[1, :, :]
    s = jnp.where(d > 0.0, lax.rsqrt(jnp.maximum(d, 1.0)), 0.0)
    s_ref[...] = s
    xs_ref[...] = s * x_ref[...]


def _dense_body(acc_ref, s_ref, x_ref, e_ref, w_ref, o_ref):
    g = s_ref[...] * (acc_ref[0, :, :] + acc_ref[1, :, :])
    xb = x_ref[...]
    y = lax.dot_general(g, w_ref[0:F, :], (((1,), (0,)), ((), ())),
                        preferred_element_type=jnp.float32)
    y = y + lax.dot_general(xb, w_ref[F:2 * F, :], (((1,), (0,)), ((), ())),
                            preferred_element_type=jnp.float32)
    out = xb
    for k in range(K):
        out = out + e_ref[:, k:k + 1] * y[:, k * F:(k + 1) * F]
    o_ref[...] = out


_BR = 1000


def kernel(x, adj, e, weights):
    row2 = adj[0].reshape(NCH, CHUNK)
    col2 = adj[1].reshape(NCH, CHUNK)
    colp = jnp.zeros((NCHP, CHUNK), jnp.int32).at[:NCH].set(col2)

    ones1 = jnp.ones((CHUNK,), jnp.float32)
    zeros1 = jnp.zeros((HWORDS,), jnp.float32)
    zeros2 = jnp.zeros((CR, F), jnp.float32)
    wall = jnp.transpose(weights, (1, 0, 2)).reshape(2 * F, K * F)

    dp = _hist_kernel(colp, ones1, zeros1).reshape(NC, NPAD)
    dp3 = dp[:, :N].reshape(NC, N, 1)

    s, xs = pl.pallas_call(
        _scale_body,
        out_shape=(
            jax.ShapeDtypeStruct((N, 1), jnp.float32),
            jax.ShapeDtypeStruct((N, F), jnp.float32),
        ),
    )(dp3, x)

    acc = _spmm_kernel(xs, row2, col2, zeros2)

    out = pl.pallas_call(
        _dense_body,
        grid=(N // _BR,),
        in_specs=[
            pl.BlockSpec((NC, _BR, F), lambda i: (0, i, 0)),
            pl.BlockSpec((_BR, 1), lambda i: (i, 0)),
            pl.BlockSpec((_BR, F), lambda i: (i, 0)),
            pl.BlockSpec((_BR, K), lambda i: (i, 0)),
            pl.BlockSpec((2 * F, K * F), lambda i: (0, 0)),
        ],
        out_specs=pl.BlockSpec((_BR, F), lambda i: (i, 0)),
        out_shape=jax.ShapeDtypeStruct((N, F), jnp.float32),
    )(acc, s, x, e, wall)
    return out
